# baseline (speedup 1.0000x reference)
import functools

import numpy as np
import jax
import jax.numpy as jnp
from jax import lax
from jax.experimental import pallas as pl
from jax.experimental.pallas import tpu as pltpu
from jax.experimental.pallas import tpu_sc as plsc

_N = 10000
_E = 160000
_H = 200
_HH = 100
_HP = 112
_R2 = 100
_C = 80
_NC = 2
_NS = 16
_CHUNKS = _E // _C
_CPT = _CHUNKS // _NS
_VPC = _C * _HH // 16
_SR = 640
_ZC = 80


def _sc_agg(x2, src2d, dst2d, et2d, nm2d, wd2, wo2, eidx, zrows):
    mesh = plsc.VectorSubcoreMesh(core_axis_name="c", subcore_axis_name="s")

    @functools.partial(
        pl.kernel,
        out_type=jax.ShapeDtypeStruct((_NC, _N, _HP), jnp.float32),
        mesh=mesh,
        scratch_types=[
            pltpu.VMEM((_R2, _HH), jnp.float32),
            pltpu.VMEM((_R2, _HH), jnp.float32),
            pltpu.VMEM((_VPC, 16), jnp.int32),
            pltpu.VMEM((_C,), jnp.int32),
            pltpu.VMEM((_C,), jnp.int32),
            pltpu.VMEM((_C,), jnp.int32),
            pltpu.VMEM((_C,), jnp.float32),
            pltpu.VMEM((_C, _HP), jnp.float32),
            pltpu.VMEM((_C, _HP), jnp.float32),
            pltpu.VMEM_SHARED((_N, _HP), jnp.float32),
            pltpu.SemaphoreType.DMA,
        ],
        compiler_params=pltpu.CompilerParams(use_tc_tiling_on_sc=False,
                                             needs_layout_passes=False),
    )
    def k(x_hbm, src_hbm, dst_hbm, et_hbm, nm_hbm, wd_hbm, wo_hbm, e_hbm,
          z_hbm, out_hbm,
          wd_v, wo_v, e_v, src_v, dst_v, et_v, nm_v, rows_v, msg_v, acc, sem):
        cid = lax.axis_index("c")
        sid = lax.axis_index("s")
        pltpu.sync_copy(wd_hbm.at[cid], wd_v)
        pltpu.sync_copy(wo_hbm.at[cid], wo_v)
        pltpu.sync_copy(e_hbm, e_v)
        pltpu.sync_copy(z_hbm, rows_v)
        pltpu.sync_copy(z_hbm, msg_v)
        iota = lax.iota(jnp.int32, 16)

        row0 = jnp.minimum(sid * _SR, _N - _SR)

        @pl.loop(0, _SR // _ZC)
        def _(i):
            off = pl.multiple_of(row0 + i * _ZC, 8)
            pltpu.sync_copy(rows_v.at[pl.ds(0, _ZC)],
                            acc.at[pl.ds(off, _ZC)])

        plsc.subcore_barrier()

        @pl.loop(0, _CPT)
        def _(kk):
            chunk = sid * _CPT + kk
            pltpu.sync_copy(src_hbm.at[chunk], src_v)
            pltpu.sync_copy(dst_hbm.at[chunk], dst_v)
            pltpu.sync_copy(et_hbm.at[chunk], et_v)
            pltpu.sync_copy(nm_hbm.at[chunk], nm_v)
            pltpu.async_copy(x_hbm.at[cid].at[src_v], rows_v, sem).wait()

            @plsc.parallel_loop(0, _VPC, unroll=8)
            def _(v):
                e = e_v[v]
                h = v * 16 + iota - e * _HH
                r = plsc.load_gather(et_v, [e])
                nm = plsc.load_gather(nm_v, [e])
                xv = plsc.load_gather(rows_v, [e, h])
                xs = plsc.load_gather(rows_v, [e, h ^ 1])
                wdv = plsc.load_gather(wd_v, [r, h])
                wov = plsc.load_gather(wo_v, [r, h])
                plsc.store_scatter(msg_v, [e, h], (xv * wdv + xs * wov) * nm)

            pltpu.sync_copy(msg_v, acc.at[dst_v], add=True)

        plsc.subcore_barrier()

        @pl.loop(0, _SR // _ZC)
        def _(i):
            off = pl.multiple_of(row0 + i * _ZC, 8)
            pltpu.sync_copy(acc.at[pl.ds(off, _ZC)],
                            out_hbm.at[cid, pl.ds(off, _ZC)])

    return k(x2, src2d, dst2d, et2d, nm2d, wd2, wo2, eidx, zrows)


def _combine(part, x, w, b, relu):
    bm = 1000

    def body(p_ref, x_ref, w_ref, b_ref, o_ref):
        acc = (p_ref[...]
               + jnp.dot(x_ref[...], w_ref[...],
                         preferred_element_type=jnp.float32)
               + b_ref[...])
        o_ref[...] = jnp.maximum(acc, 0.0) if relu else acc

    return pl.pallas_call(
        body,
        grid=(_N // bm,),
        in_specs=[
            pl.BlockSpec((bm, _H), lambda i: (i, 0)),
            pl.BlockSpec((bm, _H), lambda i: (i, 0)),
            pl.BlockSpec((_H, _H), lambda i: (0, 0)),
            pl.BlockSpec((1, _H), lambda i: (0, 0)),
        ],
        out_specs=pl.BlockSpec((bm, _H), lambda i: (i, 0)),
        out_shape=jax.ShapeDtypeStruct((_N, _H), jnp.float32),
    )(part, x, w, b.reshape(1, _H))


def _repack(W):
    wd = jnp.stack([W[:, :, 0, 0], W[:, :, 1, 1]], axis=-1).reshape(_R2, _H)
    wo = jnp.stack([W[:, :, 1, 0], W[:, :, 0, 1]], axis=-1).reshape(_R2, _H)
    wd2 = jnp.stack([wd[:, :_HH], wd[:, _HH:]])
    wo2 = jnp.stack([wo[:, :_HH], wo[:, _HH:]])
    return wd2, wo2


def _split(x):
    lo = jnp.pad(x[:, :_HH], ((0, 0), (0, _HP - _HH)))
    hi = jnp.pad(x[:, _HH:], ((0, 0), (0, _HP - _HH)))
    return jnp.stack([lo, hi])


def kernel(nids, edge_index, etypes, norm, emb, W1, loop_w1, bias1,
           W2, loop_w2, bias2):
    x = jnp.take(emb, nids, axis=0)
    src = edge_index[0].reshape(_CHUNKS, _C)
    dst = edge_index[1].reshape(_CHUNKS, _C)
    et = etypes.reshape(_CHUNKS, _C)
    nm = norm.reshape(_CHUNKS, _C)
    wd1, wo1 = _repack(W1)
    wd2, wo2 = _repack(W2)
    eidx = (jnp.arange(_C * _HH, dtype=jnp.int32) // _HH).reshape(_VPC, 16)
    zrows = jnp.zeros((_C, _HP), jnp.float32)

    p1 = _sc_agg(_split(x), src, dst, et, nm, wd1, wo1, eidx, zrows)
    part1 = jnp.concatenate([p1[0, :, :_HH], p1[1, :, :_HH]], axis=1)
    h = _combine(part1, x, loop_w1, bias1, relu=True)
    p2 = _sc_agg(_split(h), src, dst, et, nm, wd2, wo2, eidx, zrows)
    part2 = jnp.concatenate([p2[0, :, :_HH], p2[1, :, :_HH]], axis=1)
    return _combine(part2, h, loop_w2, bias2, relu=False)


# speedup vs baseline: 1.2958x; 1.2958x over previous
import functools

import numpy as np
import jax
import jax.numpy as jnp
from jax import lax
from jax.experimental import pallas as pl
from jax.experimental.pallas import tpu as pltpu
from jax.experimental.pallas import tpu_sc as plsc

_N = 10000
_E = 160000
_H = 200
_HH = 100
_HP = 112
_R2 = 100
_C = 80
_NC = 2
_NS = 16
_CHUNKS = _E // _C
_CPT = _CHUNKS // _NS
_VPC = _C * _HH // 16
_SR = 640
_ZC = 80


def _sc_agg(x2, src2d, dst2d, et2d, nm2d, wd2, wo2, eidx, zrows):
    mesh = plsc.VectorSubcoreMesh(core_axis_name="c", subcore_axis_name="s")

    @functools.partial(
        pl.kernel,
        out_type=jax.ShapeDtypeStruct((_NC, _N, _HP), jnp.float32),
        mesh=mesh,
        scratch_types=[
            pltpu.VMEM((_R2, _HH), jnp.float32),
            pltpu.VMEM((_R2, _HH), jnp.float32),
            pltpu.VMEM((_VPC, 16), jnp.int32),
            pltpu.VMEM((_C,), jnp.int32),
            pltpu.VMEM((_C,), jnp.int32),
            pltpu.VMEM((_C,), jnp.int32),
            pltpu.VMEM((_C,), jnp.float32),
            pltpu.VMEM((_C, _HP), jnp.float32),
            pltpu.VMEM((_C, _HP), jnp.float32),
            pltpu.VMEM_SHARED((_N, _HP), jnp.float32),
            pltpu.SemaphoreType.DMA,
        ],
        compiler_params=pltpu.CompilerParams(use_tc_tiling_on_sc=False,
                                             needs_layout_passes=False),
    )
    def k(x_hbm, src_hbm, dst_hbm, et_hbm, nm_hbm, wd_hbm, wo_hbm, e_hbm,
          z_hbm, out_hbm,
          wd_v, wo_v, e_v, src_v, dst_v, et_v, nm_v, rows_v, msg_v, acc, sem):
        cid = lax.axis_index("c")
        sid = lax.axis_index("s")
        pltpu.sync_copy(wd_hbm.at[cid], wd_v)
        pltpu.sync_copy(wo_hbm.at[cid], wo_v)
        pltpu.sync_copy(e_hbm, e_v)
        pltpu.sync_copy(z_hbm, rows_v)
        pltpu.sync_copy(z_hbm, msg_v)
        iota = lax.iota(jnp.int32, 16)

        row0 = jnp.minimum(sid * _SR, _N - _SR)

        @pl.loop(0, _SR // _ZC)
        def _(i):
            off = pl.multiple_of(row0 + i * _ZC, 8)
            pltpu.sync_copy(rows_v.at[pl.ds(0, _ZC)],
                            acc.at[pl.ds(off, _ZC)])

        plsc.subcore_barrier()

        @pl.loop(0, _CPT)
        def _(kk):
            chunk = sid * _CPT + kk
            c1 = pltpu.async_copy(src_hbm.at[chunk], src_v, sem)
            c2 = pltpu.async_copy(dst_hbm.at[chunk], dst_v, sem)
            c3 = pltpu.async_copy(et_hbm.at[chunk], et_v, sem)
            c4 = pltpu.async_copy(nm_hbm.at[chunk], nm_v, sem)
            c1.wait(); c2.wait(); c3.wait(); c4.wait()
            pltpu.async_copy(x_hbm.at[cid].at[src_v], rows_v, sem).wait()

            @plsc.parallel_loop(0, _VPC, unroll=4)
            def _(v):
                e = e_v[v]
                h = v * 16 + iota - e * _HH
                r = plsc.load_gather(et_v, [e])
                nm = plsc.load_gather(nm_v, [e])
                xv = plsc.load_gather(rows_v, [e, h])
                xs = plsc.load_gather(rows_v, [e, h ^ 1])
                wdv = plsc.load_gather(wd_v, [r, h])
                wov = plsc.load_gather(wo_v, [r, h])
                plsc.store_scatter(msg_v, [e, h], (xv * wdv + xs * wov) * nm)

            pltpu.sync_copy(msg_v, acc.at[dst_v], add=True)

        plsc.subcore_barrier()

        @pl.loop(0, _SR // _ZC)
        def _(i):
            off = pl.multiple_of(row0 + i * _ZC, 8)
            pltpu.sync_copy(acc.at[pl.ds(off, _ZC)],
                            out_hbm.at[cid, pl.ds(off, _ZC)])

    return k(x2, src2d, dst2d, et2d, nm2d, wd2, wo2, eidx, zrows)


def _combine(part, x, w, b, relu):
    bm = 1000

    def body(p_ref, x_ref, w_ref, b_ref, o_ref):
        acc = (p_ref[...]
               + jnp.dot(x_ref[...], w_ref[...],
                         preferred_element_type=jnp.float32)
               + b_ref[...])
        o_ref[...] = jnp.maximum(acc, 0.0) if relu else acc

    return pl.pallas_call(
        body,
        grid=(_N // bm,),
        in_specs=[
            pl.BlockSpec((bm, _H), lambda i: (i, 0)),
            pl.BlockSpec((bm, _H), lambda i: (i, 0)),
            pl.BlockSpec((_H, _H), lambda i: (0, 0)),
            pl.BlockSpec((1, _H), lambda i: (0, 0)),
        ],
        out_specs=pl.BlockSpec((bm, _H), lambda i: (i, 0)),
        out_shape=jax.ShapeDtypeStruct((_N, _H), jnp.float32),
    )(part, x, w, b.reshape(1, _H))


def _repack(W):
    wd = jnp.stack([W[:, :, 0, 0], W[:, :, 1, 1]], axis=-1).reshape(_R2, _H)
    wo = jnp.stack([W[:, :, 1, 0], W[:, :, 0, 1]], axis=-1).reshape(_R2, _H)
    wd2 = jnp.stack([wd[:, :_HH], wd[:, _HH:]])
    wo2 = jnp.stack([wo[:, :_HH], wo[:, _HH:]])
    return wd2, wo2


def _split(x):
    lo = jnp.pad(x[:, :_HH], ((0, 0), (0, _HP - _HH)))
    hi = jnp.pad(x[:, _HH:], ((0, 0), (0, _HP - _HH)))
    return jnp.stack([lo, hi])


def kernel(nids, edge_index, etypes, norm, emb, W1, loop_w1, bias1,
           W2, loop_w2, bias2):
    x = jnp.take(emb, nids, axis=0)
    src = edge_index[0].reshape(_CHUNKS, _C)
    dst = edge_index[1].reshape(_CHUNKS, _C)
    et = etypes.reshape(_CHUNKS, _C)
    nm = norm.reshape(_CHUNKS, _C)
    wd1, wo1 = _repack(W1)
    wd2, wo2 = _repack(W2)
    eidx = (jnp.arange(_C * _HH, dtype=jnp.int32) // _HH).reshape(_VPC, 16)
    zrows = jnp.zeros((_C, _HP), jnp.float32)

    p1 = _sc_agg(_split(x), src, dst, et, nm, wd1, wo1, eidx, zrows)
    part1 = jnp.concatenate([p1[0, :, :_HH], p1[1, :, :_HH]], axis=1)
    h = _combine(part1, x, loop_w1, bias1, relu=True)
    p2 = _sc_agg(_split(h), src, dst, et, nm, wd2, wo2, eidx, zrows)
    part2 = jnp.concatenate([p2[0, :, :_HH], p2[1, :, :_HH]], axis=1)
    return _combine(part2, h, loop_w2, bias2, relu=False)
